# baseline (speedup 1.0000x reference)
import jax
import jax.numpy as jnp
from jax.experimental import pallas as pl
from jax.experimental.pallas import tpu as pltpu

_EPS = 1.0


def _fused_body(z_ref, nmu_ref, onesk_ref, d2c_ref, mass_ref, mut_ref,
                ones1k_ref, wt_ref, b_ref, ones1d_ref, ones1dc_ref,
                fine_ref, coarse_ref, comb_ref):
    zt = z_ref[...]
    d = zt.shape[0]

    d2e = (jnp.dot(nmu_ref[...], zt, preferred_element_type=jnp.float32)
           + jnp.dot(onesk_ref[...], zt * zt,
                     preferred_element_type=jnp.float32)
           + d2c_ref[...])
    w = mass_ref[...] / d2e

    wmu = jnp.dot(mut_ref[...], w, preferred_element_type=jnp.float32)
    sw = jnp.dot(ones1k_ref[...], w, preferred_element_type=jnp.float32)
    fine = zt * (1.0 - sw) + wmu

    cr = (jnp.dot(wt_ref[...], fine, preferred_element_type=jnp.float32)
          + b_ref[...])

    ssf = jnp.dot(ones1d_ref[...], fine * fine,
                  preferred_element_type=jnp.float32)
    ssc = jnp.dot(ones1dc_ref[...], cr * cr,
                  preferred_element_type=jnp.float32)
    inv_f = jax.lax.rsqrt(ssf + 1e-30)
    inv_c = jax.lax.rsqrt(ssc + 1e-30)

    comb = jnp.concatenate([fine * inv_f, cr * inv_c], axis=0)
    fine_ref[...] = fine
    coarse_ref[...] = comb[d:]
    comb_ref[...] = comb


@jax.jit
def _run(z, mu_fine, mass_fine, W_coarse, b_coarse):
    n, d = z.shape
    k = mu_fine.shape[0]
    dc = W_coarse.shape[1]
    block = 28672
    grid = (pl.cdiv(n, block),)

    mu = mu_fine
    nmu = -2.0 * mu
    onesk = jnp.ones((k, d), jnp.float32)
    d2c = (jnp.sum(mu * mu, axis=1) + _EPS)[:, None]
    mass = mass_fine[:, None]
    mut = mu.T
    ones1k = jnp.ones((1, k), jnp.float32)
    ones1d = jnp.ones((1, d), jnp.float32)
    ones1dc = jnp.ones((1, dc), jnp.float32)

    full = lambda shape: pl.BlockSpec(shape, lambda i: (0, 0))
    finet, coarset, combt = pl.pallas_call(
        _fused_body,
        grid=grid,
        in_specs=[
            pl.BlockSpec((d, block), lambda i: (0, i)),
            full((k, d)), full((k, d)), full((k, 1)), full((k, 1)),
            full((d, k)), full((1, k)), full((dc, d)), full((dc, 1)),
            full((1, d)), full((1, dc)),
        ],
        out_specs=[
            pl.BlockSpec((d, block), lambda i: (0, i)),
            pl.BlockSpec((dc, block), lambda i: (0, i)),
            pl.BlockSpec((d + dc, block), lambda i: (0, i)),
        ],
        out_shape=[
            jax.ShapeDtypeStruct((d, n), jnp.float32),
            jax.ShapeDtypeStruct((dc, n), jnp.float32),
            jax.ShapeDtypeStruct((d + dc, n), jnp.float32),
        ],
        compiler_params=pltpu.CompilerParams(
            dimension_semantics=("parallel",)),
    )(z.T, nmu, onesk, d2c, mass, mut, ones1k, W_coarse.T, b_coarse[:, None],
      ones1d, ones1dc)
    return finet.T, coarset.T, combt.T


def kernel(z, mu_fine, mass_fine, W_coarse, b_coarse):
    return _run(z, mu_fine, mass_fine, W_coarse, b_coarse)
